# baseline (speedup 1.0000x reference)
import functools

import jax
import jax.numpy as jnp
from jax import lax
from jax.experimental import pallas as pl
from jax.experimental.pallas import tpu as pltpu
from jax.experimental.pallas import tpu_sc as plsc

jax.config.update("jax_use_simplified_jaxpr_constants", True)


def _enable_const_args():
    import inspect
    from jax._src.interpreters import mlir as _mlir
    lp = _mlir.LoweringParameters
    names = [p for p in inspect.signature(lp.__init__).parameters
             if p != "self"]
    idx = names.index("hoist_constants_as_args")
    defaults = list(lp.__init__.__defaults__)
    defaults[idx] = True
    lp.__init__.__defaults__ = tuple(defaults)


_enable_const_args()

_MARGIN = 0.2
_NC, _NS, _L = 2, 16, 16
_NW = _NC * _NS
_RB = 4
_TCR = 256
_SC_FRAC = 0.25

_NEG_INF = float("-inf")
_BIG = 1 << 30


@functools.cache
def _gumbel_consts(b):
    with jax.ensure_compile_time_eval():
        kp, kn = jax.random.split(jax.random.key(42))
        gp = jax.random.gumbel(kp, (b, b), jnp.float32)
        gn = jax.random.gumbel(kn, (b, b), jnp.float32)
    return gp, gn


def _tc_body(pids_row_ref, pids_all_ref, cdist_ref, gp_ref, gn_ref, out_ref):
    cd = cdist_ref[...]
    r, b = cd.shape
    mask = pids_row_ref[...][:, None] == pids_all_ref[...][None, :]
    neg_inf = jnp.float32(_NEG_INF)
    p = jnp.where(mask, cd, neg_inf) + gp_ref[...]
    n = jnp.where(mask, neg_inf, -cd) + gn_ref[...]
    iota = lax.broadcasted_iota(jnp.int32, (r, b), 1)
    pmax = jnp.max(p, axis=1, keepdims=True)
    ipos = jnp.min(jnp.where(p == pmax, iota, b), axis=1, keepdims=True)
    nmax = jnp.max(n, axis=1, keepdims=True)
    ineg = jnp.min(jnp.where(n == nmax, iota, b), axis=1, keepdims=True)
    vpos = jnp.max(jnp.where(iota == ipos, cd, neg_inf), axis=1)
    vneg = jnp.max(jnp.where(iota == ineg, cd, neg_inf), axis=1)
    out_ref[...] = jnp.maximum(vpos - vneg + jnp.float32(_MARGIN), 0.0)


@functools.cache
def _tc_kernel_cached(b, nrows):
    r = min(_TCR, nrows)
    return pl.pallas_call(
        _tc_body,
        grid=(nrows // r,),
        in_specs=[
            pl.BlockSpec((r,), lambda i: (i,)),
            pl.BlockSpec((b,), lambda i: (0,)),
            pl.BlockSpec((r, b), lambda i: (i, 0)),
            pl.BlockSpec((r, b), lambda i: (i, 0)),
            pl.BlockSpec((r, b), lambda i: (i, 0)),
        ],
        out_specs=pl.BlockSpec((r,), lambda i: (i,)),
        out_shape=jax.ShapeDtypeStruct((nrows,), jnp.float32),
    )


def _row_sample(pids_v, cd_t, gp_t, gn_t, r, i, b):
    lane = lax.broadcasted_iota(jnp.int32, (_L,), 0)
    pidvec = plsc.load_gather(pids_v, [jnp.full((_L,), i, jnp.int32)])
    ninf = jnp.full((_L,), _NEG_INF, jnp.float32)

    @plsc.parallel_loop(0, b // _L, unroll=8,
                        carry=(ninf, lane, ninf, lane, lane))
    def chunk(c, carry):
        cmaxp, cidxp, cmaxn, cidxn, idxv = carry
        sl = pl.ds(c * _L, _L)
        cd = cd_t[r, sl]
        m = pids_v[sl] == pidvec
        p = jnp.where(m, cd + gp_t[r, sl], ninf)
        n = jnp.where(m, ninf, gn_t[r, sl] - cd)
        up = p > cmaxp
        cmaxp = jnp.where(up, p, cmaxp)
        cidxp = jnp.where(up, idxv, cidxp)
        un = n > cmaxn
        cmaxn = jnp.where(un, n, cmaxn)
        cidxn = jnp.where(un, idxv, cidxn)
        return cmaxp, cidxp, cmaxn, cidxn, idxv + _L

    cmaxp, cidxp, cmaxn, cidxn, _ = chunk
    rvec = jnp.full((_L,), r, jnp.int32)
    gip = jnp.min(jnp.where(cmaxp == jnp.max(cmaxp), cidxp, _BIG))
    gin = jnp.min(jnp.where(cmaxn == jnp.max(cmaxn), cidxn, _BIG))
    vpos = plsc.load_gather(cd_t, [rvec, jnp.full((_L,), gip, jnp.int32)])
    vneg = plsc.load_gather(cd_t, [rvec, jnp.full((_L,), gin, jnp.int32)])
    return jnp.maximum(vpos - vneg + jnp.float32(_MARGIN), 0.0)


def _make_sc_kernel(b, s):
    rows_per_w = s // _NW
    nbatch = rows_per_w // _RB
    row0 = b - s
    mesh = plsc.VectorSubcoreMesh(core_axis_name="c", subcore_axis_name="s")

    @functools.partial(
        pl.kernel,
        out_type=jax.ShapeDtypeStruct((s,), jnp.float32),
        mesh=mesh,
        compiler_params=pltpu.CompilerParams(needs_layout_passes=False),
        scratch_types=[
            pltpu.VMEM((b,), jnp.int32),
            pltpu.VMEM((_RB, b), jnp.float32),
            pltpu.VMEM((_RB, b), jnp.float32),
            pltpu.VMEM((_RB, b), jnp.float32),
            pltpu.VMEM((_RB, b), jnp.float32),
            pltpu.VMEM((_RB, b), jnp.float32),
            pltpu.VMEM((_RB, b), jnp.float32),
            pltpu.VMEM((rows_per_w,), jnp.float32),
            pltpu.SemaphoreType.DMA,
            pltpu.SemaphoreType.DMA,
        ],
    )
    def sc_kernel(cdist_hbm, pids_hbm, gp_hbm, gn_hbm, out_hbm,
                  pids_v, cd_a, cd_b, gp_a, gp_b, gn_a, gn_b, out_v,
                  sem_a, sem_b):
        wid = lax.axis_index("s") * _NC + lax.axis_index("c")
        lbase = wid * rows_per_w
        base = row0 + lbase
        pltpu.sync_copy(pids_hbm, pids_v)

        def issue(i, cd_t, gp_t, gn_t, sem):
            sl = pl.ds(i, _RB)
            pltpu.async_copy(cdist_hbm.at[sl], cd_t, sem)
            pltpu.async_copy(gp_hbm.at[sl], gp_t, sem)
            pltpu.async_copy(gn_hbm.at[sl], gn_t, sem)

        def wait(cd_t, gp_t, gn_t, sem):
            pltpu.make_async_copy(cdist_hbm.at[pl.ds(0, _RB)], cd_t, sem).wait()
            pltpu.make_async_copy(gp_hbm.at[pl.ds(0, _RB)], gp_t, sem).wait()
            pltpu.make_async_copy(gn_hbm.at[pl.ds(0, _RB)], gn_t, sem).wait()

        lane = lax.broadcasted_iota(jnp.int32, (_L,), 0)
        mask0 = lane == 0

        def rows(i0, r0, cd_t, gp_t, gn_t):
            for r in range(_RB):
                dv = _row_sample(pids_v, cd_t, gp_t, gn_t, r, i0 + r, b)
                plsc.store_scatter(out_v, [jnp.full((_L,), r0 + r, jnp.int32)],
                                   dv, mask=mask0)

        issue(base, cd_a, gp_a, gn_a, sem_a)

        def two_batches(t, _):
            r0 = 2 * t * _RB
            i0 = base + r0
            issue(i0 + _RB, cd_b, gp_b, gn_b, sem_b)
            wait(cd_a, gp_a, gn_a, sem_a)
            rows(i0, r0, cd_a, gp_a, gn_a)
            inext = jnp.minimum(i0 + 2 * _RB, base + rows_per_w - _RB)
            issue(inext, cd_a, gp_a, gn_a, sem_a)
            wait(cd_b, gp_b, gn_b, sem_b)
            rows(i0 + _RB, r0 + _RB, cd_b, gp_b, gn_b)
            return 0

        lax.fori_loop(0, nbatch // 2, two_batches, 0)
        wait(cd_a, gp_a, gn_a, sem_a)
        pltpu.sync_copy(out_v, out_hbm.at[pl.ds(lbase, rows_per_w)])

    return sc_kernel


@functools.cache
def _sc_kernel_cached(b, s):
    return _make_sc_kernel(b, s)


def _pick_split(b):
    q = _NW * _RB * 2
    s = int(b * _SC_FRAC) // q * q
    while s > 0 and ((b - s) % _TCR or (s % q)):
        s -= q
    return s


def kernel(cdist, pids):
    b = cdist.shape[0]
    gp, gn = _gumbel_consts(b)
    s = _pick_split(b)
    if s == 0 or b % _NW:
        return _tc_kernel_cached(b, b)(pids, pids, cdist, gp, gn)
    sc_out = _sc_kernel_cached(b, s)(cdist, pids, gp, gn)
    tc_out = _tc_kernel_cached(b, b - s)(pids, pids, cdist, gp, gn)
    return jnp.concatenate([tc_out, sc_out])


# speedup vs baseline: 1.0122x; 1.0122x over previous
import functools

import jax
import jax.numpy as jnp
from jax import lax
from jax.experimental import pallas as pl
from jax.experimental.pallas import tpu as pltpu
from jax.experimental.pallas import tpu_sc as plsc

jax.config.update("jax_use_simplified_jaxpr_constants", True)


def _enable_const_args():
    import inspect
    from jax._src.interpreters import mlir as _mlir
    lp = _mlir.LoweringParameters
    names = [p for p in inspect.signature(lp.__init__).parameters
             if p != "self"]
    idx = names.index("hoist_constants_as_args")
    defaults = list(lp.__init__.__defaults__)
    defaults[idx] = True
    lp.__init__.__defaults__ = tuple(defaults)


_enable_const_args()

_MARGIN = 0.2
_NC, _NS, _L = 2, 16, 16
_NW = _NC * _NS
_RB = 4
_TCR = 256
_SC_FRAC = 0.5

_NEG_INF = float("-inf")
_BIG = 1 << 30


@functools.cache
def _gumbel_consts(b):
    with jax.ensure_compile_time_eval():
        kp, kn = jax.random.split(jax.random.key(42))
        gp = jax.random.gumbel(kp, (b, b), jnp.float32)
        gn = jax.random.gumbel(kn, (b, b), jnp.float32)
    return gp, gn


def _tc_body(pids_row_ref, pids_all_ref, cdist_ref, gp_ref, gn_ref, out_ref):
    cd = cdist_ref[...]
    r, b = cd.shape
    mask = pids_row_ref[...][:, None] == pids_all_ref[...][None, :]
    neg_inf = jnp.float32(_NEG_INF)
    p = jnp.where(mask, cd, neg_inf) + gp_ref[...]
    n = jnp.where(mask, neg_inf, -cd) + gn_ref[...]
    iota = lax.broadcasted_iota(jnp.int32, (r, b), 1)
    pmax = jnp.max(p, axis=1, keepdims=True)
    ipos = jnp.min(jnp.where(p == pmax, iota, b), axis=1, keepdims=True)
    nmax = jnp.max(n, axis=1, keepdims=True)
    ineg = jnp.min(jnp.where(n == nmax, iota, b), axis=1, keepdims=True)
    vpos = jnp.max(jnp.where(iota == ipos, cd, neg_inf), axis=1)
    vneg = jnp.max(jnp.where(iota == ineg, cd, neg_inf), axis=1)
    out_ref[...] = jnp.maximum(vpos - vneg + jnp.float32(_MARGIN), 0.0)


@functools.cache
def _tc_kernel_cached(b, nrows):
    r = min(_TCR, nrows)
    return pl.pallas_call(
        _tc_body,
        grid=(nrows // r,),
        in_specs=[
            pl.BlockSpec((r,), lambda i: (i,)),
            pl.BlockSpec((b,), lambda i: (0,)),
            pl.BlockSpec((r, b), lambda i: (i, 0)),
            pl.BlockSpec((r, b), lambda i: (i, 0)),
            pl.BlockSpec((r, b), lambda i: (i, 0)),
        ],
        out_specs=pl.BlockSpec((r,), lambda i: (i,)),
        out_shape=jax.ShapeDtypeStruct((nrows,), jnp.float32),
    )


def _row_sample(pids_v, cd_t, gp_t, gn_t, r, i, b):
    lane = lax.broadcasted_iota(jnp.int32, (_L,), 0)
    pidvec = plsc.load_gather(pids_v, [jnp.full((_L,), i, jnp.int32)])
    ninf = jnp.full((_L,), _NEG_INF, jnp.float32)

    @plsc.parallel_loop(0, b // _L, unroll=8,
                        carry=(ninf, lane, ninf, lane, lane))
    def chunk(c, carry):
        cmaxp, cidxp, cmaxn, cidxn, idxv = carry
        sl = pl.ds(c * _L, _L)
        cd = cd_t[r, sl]
        m = pids_v[sl] == pidvec
        p = jnp.where(m, cd + gp_t[r, sl], ninf)
        n = jnp.where(m, ninf, gn_t[r, sl] - cd)
        up = p > cmaxp
        cmaxp = jnp.where(up, p, cmaxp)
        cidxp = jnp.where(up, idxv, cidxp)
        un = n > cmaxn
        cmaxn = jnp.where(un, n, cmaxn)
        cidxn = jnp.where(un, idxv, cidxn)
        return cmaxp, cidxp, cmaxn, cidxn, idxv + _L

    cmaxp, cidxp, cmaxn, cidxn, _ = chunk
    rvec = jnp.full((_L,), r, jnp.int32)
    gip = jnp.min(jnp.where(cmaxp == jnp.max(cmaxp), cidxp, _BIG))
    gin = jnp.min(jnp.where(cmaxn == jnp.max(cmaxn), cidxn, _BIG))
    vpos = plsc.load_gather(cd_t, [rvec, jnp.full((_L,), gip, jnp.int32)])
    vneg = plsc.load_gather(cd_t, [rvec, jnp.full((_L,), gin, jnp.int32)])
    return jnp.maximum(vpos - vneg + jnp.float32(_MARGIN), 0.0)


def _make_sc_kernel(b, s):
    rows_per_w = s // _NW
    nbatch = rows_per_w // _RB
    row0 = b - s
    mesh = plsc.VectorSubcoreMesh(core_axis_name="c", subcore_axis_name="s")

    @functools.partial(
        pl.kernel,
        out_type=jax.ShapeDtypeStruct((s,), jnp.float32),
        mesh=mesh,
        compiler_params=pltpu.CompilerParams(needs_layout_passes=False),
        scratch_types=[
            pltpu.VMEM((b,), jnp.int32),
            pltpu.VMEM((_RB, b), jnp.float32),
            pltpu.VMEM((_RB, b), jnp.float32),
            pltpu.VMEM((_RB, b), jnp.float32),
            pltpu.VMEM((_RB, b), jnp.float32),
            pltpu.VMEM((_RB, b), jnp.float32),
            pltpu.VMEM((_RB, b), jnp.float32),
            pltpu.VMEM((rows_per_w,), jnp.float32),
            pltpu.SemaphoreType.DMA,
            pltpu.SemaphoreType.DMA,
        ],
    )
    def sc_kernel(cdist_hbm, pids_hbm, gp_hbm, gn_hbm, out_hbm,
                  pids_v, cd_a, cd_b, gp_a, gp_b, gn_a, gn_b, out_v,
                  sem_a, sem_b):
        wid = lax.axis_index("s") * _NC + lax.axis_index("c")
        lbase = wid * rows_per_w
        base = row0 + lbase
        pltpu.sync_copy(pids_hbm, pids_v)

        def issue(i, cd_t, gp_t, gn_t, sem):
            sl = pl.ds(i, _RB)
            pltpu.async_copy(cdist_hbm.at[sl], cd_t, sem)
            pltpu.async_copy(gp_hbm.at[sl], gp_t, sem)
            pltpu.async_copy(gn_hbm.at[sl], gn_t, sem)

        def wait(cd_t, gp_t, gn_t, sem):
            pltpu.make_async_copy(cdist_hbm.at[pl.ds(0, _RB)], cd_t, sem).wait()
            pltpu.make_async_copy(gp_hbm.at[pl.ds(0, _RB)], gp_t, sem).wait()
            pltpu.make_async_copy(gn_hbm.at[pl.ds(0, _RB)], gn_t, sem).wait()

        lane = lax.broadcasted_iota(jnp.int32, (_L,), 0)
        mask0 = lane == 0

        def rows(i0, r0, cd_t, gp_t, gn_t):
            for r in range(_RB):
                dv = _row_sample(pids_v, cd_t, gp_t, gn_t, r, i0 + r, b)
                plsc.store_scatter(out_v, [jnp.full((_L,), r0 + r, jnp.int32)],
                                   dv, mask=mask0)

        issue(base, cd_a, gp_a, gn_a, sem_a)

        def two_batches(t, _):
            r0 = 2 * t * _RB
            i0 = base + r0
            issue(i0 + _RB, cd_b, gp_b, gn_b, sem_b)
            wait(cd_a, gp_a, gn_a, sem_a)
            rows(i0, r0, cd_a, gp_a, gn_a)
            inext = jnp.minimum(i0 + 2 * _RB, base + rows_per_w - _RB)
            issue(inext, cd_a, gp_a, gn_a, sem_a)
            wait(cd_b, gp_b, gn_b, sem_b)
            rows(i0 + _RB, r0 + _RB, cd_b, gp_b, gn_b)
            return 0

        lax.fori_loop(0, nbatch // 2, two_batches, 0)
        wait(cd_a, gp_a, gn_a, sem_a)
        pltpu.sync_copy(out_v, out_hbm.at[pl.ds(lbase, rows_per_w)])

    return sc_kernel


@functools.cache
def _sc_kernel_cached(b, s):
    return _make_sc_kernel(b, s)


def _pick_split(b):
    q = _NW * _RB * 2
    s = int(b * _SC_FRAC) // q * q
    while s > 0 and ((b - s) % _TCR or (s % q)):
        s -= q
    return s


def kernel(cdist, pids):
    b = cdist.shape[0]
    gp, gn = _gumbel_consts(b)
    s = _pick_split(b)
    if s == 0 or b % _NW:
        return _tc_kernel_cached(b, b)(pids, pids, cdist, gp, gn)
    sc_out = _sc_kernel_cached(b, s)(cdist, pids, gp, gn)
    tc_out = _tc_kernel_cached(b, b - s)(pids, pids, cdist, gp, gn)
    return jnp.concatenate([tc_out, sc_out])
